# baseline (speedup 1.0000x reference)
import functools

import numpy as np

import jax
import jax.numpy as jnp
from jax.experimental import pallas as pl
from jax.experimental.pallas import tpu as pltpu

_NBR = 5
_K = 1
_NBC = _NBR // _K


def _build_schedule(nbr, k):
    nbc = nbr // k
    comp = lambda c: (c + 1) * k - 1
    rows, cols, fin_s2, ph1, wout, rstart = [], [], [], [], [], []
    passb = []
    for r in range(nbr):
        later = [c for c in range(nbc) if comp(c) > r]
        ready = [c for c in range(nbc) if comp(c) < r]
        trick = [c for c in range(nbc) if comp(c) == r]
        order = later + ready + trick
        for j, c in enumerate(order):
            rows.append(r)
            cols.append(c)
            rstart.append(1 if j == 0 else 0)
            last = j == len(order) - 1
            fin_s2.append(1 if last else 0)
            eligible = c in ready or (c in trick and last)
            ph1.append(1 if eligible else 0)
            if not eligible:
                passb.append((r, c))
            wout.append(0)
    b_by_row = {}
    for r, c in passb:
        b_by_row.setdefault(r, []).append(c)
    for r in range(nbr):
        for j, c in enumerate(sorted(b_by_row.get(r, []))):
            rows.append(r)
            cols.append(c)
            rstart.append(0)
            fin_s2.append(0)
            ph1.append(1)
            wout.append(1 if j == len(b_by_row[r]) - 1 else 0)
    t_a = nbr * nbc
    for r in range(nbr):
        if r not in b_by_row:
            for t in range(t_a):
                if rows[t] == r and fin_s2[t]:
                    wout[t] = 1
    t_total = len(rows)
    seen = set()
    ph1f = [0] * t_total
    for t in range(t_total):
        if ph1[t] and rows[t] not in seen:
            seen.add(rows[t])
            ph1f[t] = 1
    rout = [0] * t_total
    nxt = rows[-1]
    for t in range(t_total - 1, -1, -1):
        if wout[t]:
            nxt = rows[t]
        rout[t] = nxt
    mk = lambda a: np.asarray(a, dtype=np.int32)
    return (mk(rows), mk(cols), mk(rout), mk(rstart), mk(fin_s2), mk(ph1),
            mk(ph1f), mk(wout)), t_a


_SCHED, _TA = _build_schedule(_NBR, _K)
_T = int(_SCHED[0].shape[0])


def _xw_kernel(x_ref, w_ref, o_ref):
    n = x_ref.shape[0]
    o_ref[...] = jnp.zeros_like(o_ref)
    o_ref[pl.ds(0, n), :] = jnp.dot(
        x_ref[...], w_ref[...],
        preferred_element_type=jnp.float32).astype(jnp.bfloat16)


def _gcn_kernel(n_valid, rows_ref, cols_ref, rout_ref, rstart_ref, fin_ref,
                ph1_ref, ph1f_ref, wout_ref, cmask_ref, adj_ref, s1_ref,
                b1_ref, w2_ref, b2_ref, o_ref, h_ref, s2_ref, oacc_ref):
    t = pl.program_id(0)
    bm = adj_ref.shape[0]
    w = adj_ref.shape[1]
    r = rows_ref[t]
    c = cols_ref[t]
    valid_last = n_valid - (_NBC - 1) * w

    if valid_last < w:

        @pl.when(cmask_ref[t] == 1)
        def _zero_overhang_cols():
            adj_ref[:, pl.ds(valid_last, w - valid_last)] = jnp.zeros(
                (bm, w - valid_last), jnp.float32)

    @pl.when(rstart_ref[t] == 1)
    def _row_start():
        h_ref[...] = jnp.zeros_like(h_ref)

    adj_bf = adj_ref[...].astype(jnp.bfloat16)

    @pl.when(t < _TA)
    def _phase0():
        h_ref[...] += jnp.dot(adj_bf, s1_ref[pl.ds(c * w, w), :],
                              preferred_element_type=jnp.float32)

    @pl.when(fin_ref[t] == 1)
    def _finalize_s2():
        h = jnp.maximum(h_ref[...] + b1_ref[...], 0.0)
        s2_blk = jnp.dot(h, w2_ref[...], preferred_element_type=jnp.float32)
        row_ids = r * bm + jax.lax.broadcasted_iota(jnp.int32, s2_blk.shape,
                                                    0)
        s2_ref[pl.ds(r * bm, bm), :] = jnp.where(
            row_ids < n_valid, s2_blk, 0.0).astype(jnp.bfloat16)

    @pl.when(ph1_ref[t] == 1)
    def _phase1():
        contrib = jnp.dot(adj_bf, s2_ref[pl.ds(c * w, w), :],
                          preferred_element_type=jnp.float32)

        @pl.when(ph1f_ref[t] == 1)
        def _first():
            oacc_ref[pl.ds(r * bm, bm), :] = contrib

        @pl.when(ph1f_ref[t] == 0)
        def _rest():
            oacc_ref[pl.ds(r * bm, bm), :] += contrib

    @pl.when(wout_ref[t] == 1)
    def _writeout():
        o = oacc_ref[pl.ds(r * bm, bm), :] + b2_ref[...]
        m = jnp.max(o, axis=-1, keepdims=True)
        e = o - m
        lse = jnp.log(jnp.sum(jnp.exp(e), axis=-1, keepdims=True))
        o_ref[...] = e - lse


def kernel(x, adj, W1, b1, W2, b2):
    n, _ = x.shape
    hid = W1.shape[1]
    out_f = W2.shape[1]
    per_blk = (n + _NBR - 1) // _NBR
    bm = ((per_blk + 127) // 128) * 128
    w = _K * bm
    npad = _NBR * bm

    s1p = pl.pallas_call(
        _xw_kernel,
        out_shape=jax.ShapeDtypeStruct((npad, hid), jnp.bfloat16),
    )(x, W1)

    b1r = b1.reshape(1, hid)
    b2r = b2.reshape(1, out_f)
    cmask = ((_SCHED[1] == _NBC - 1) & (n % w != 0)).astype(np.int32)
    sched = tuple(jnp.asarray(a) for a in _SCHED) + (jnp.asarray(cmask),)

    grid_spec = pltpu.PrefetchScalarGridSpec(
        num_scalar_prefetch=9,
        grid=(_T,),
        in_specs=[
            pl.BlockSpec((bm, w),
                         lambda t, rows, cols, *_: (rows[t], cols[t])),
            pl.BlockSpec((npad, hid), lambda t, *_: (0, 0)),
            pl.BlockSpec((1, hid), lambda t, *_: (0, 0)),
            pl.BlockSpec((hid, out_f), lambda t, *_: (0, 0)),
            pl.BlockSpec((1, out_f), lambda t, *_: (0, 0)),
        ],
        out_specs=pl.BlockSpec((bm, out_f),
                               lambda t, rows, cols, rout, *_: (rout[t], 0)),
        scratch_shapes=[
            pltpu.VMEM((bm, hid), jnp.float32),
            pltpu.VMEM((npad, out_f), jnp.bfloat16),
            pltpu.VMEM((npad, out_f), jnp.float32),
        ],
    )

    out = pl.pallas_call(
        functools.partial(_gcn_kernel, n),
        grid_spec=grid_spec,
        out_shape=jax.ShapeDtypeStruct((n, out_f), jnp.float32),
    )(*sched, adj, s1p, b1r, W2, b2r)
    return out


# speedup vs baseline: 1.0800x; 1.0800x over previous
import functools

import numpy as np

import jax
import jax.numpy as jnp
from jax.experimental import pallas as pl
from jax.experimental.pallas import tpu as pltpu

_NBR = 5
_K = 1
_NBC = _NBR // _K


def _build_schedule(nbr, k):
    nbc = nbr // k
    comp = lambda c: (c + 1) * k - 1
    rows, cols, fin_s2, ph1, wout, rstart = [], [], [], [], [], []
    passb = []
    for r in range(nbr):
        later = [c for c in range(nbc) if comp(c) > r]
        ready = [c for c in range(nbc) if comp(c) < r]
        trick = [c for c in range(nbc) if comp(c) == r]
        order = later + ready + trick
        for j, c in enumerate(order):
            rows.append(r)
            cols.append(c)
            rstart.append(1 if j == 0 else 0)
            last = j == len(order) - 1
            fin_s2.append(1 if last else 0)
            eligible = c in ready or (c in trick and last)
            ph1.append(1 if eligible else 0)
            if not eligible:
                passb.append((r, c))
            wout.append(0)
    b_by_row = {}
    for r, c in passb:
        b_by_row.setdefault(r, []).append(c)
    for r in range(nbr):
        for j, c in enumerate(sorted(b_by_row.get(r, []))):
            rows.append(r)
            cols.append(c)
            rstart.append(0)
            fin_s2.append(0)
            ph1.append(1)
            wout.append(1 if j == len(b_by_row[r]) - 1 else 0)
    t_a = nbr * nbc
    for r in range(nbr):
        if r not in b_by_row:
            for t in range(t_a):
                if rows[t] == r and fin_s2[t]:
                    wout[t] = 1
    t_total = len(rows)
    seen = set()
    ph1f = [0] * t_total
    for t in range(t_total):
        if ph1[t] and rows[t] not in seen:
            seen.add(rows[t])
            ph1f[t] = 1
    rout = [0] * t_total
    nxt = rows[-1]
    for t in range(t_total - 1, -1, -1):
        if wout[t]:
            nxt = rows[t]
        rout[t] = nxt
    mk = lambda a: np.asarray(a, dtype=np.int32)
    return (mk(rows), mk(cols), mk(rout), mk(rstart), mk(fin_s2), mk(ph1),
            mk(ph1f), mk(wout)), t_a


_SCHED, _TA = _build_schedule(_NBR, _K)
_T = int(_SCHED[0].shape[0])


def _gcn_kernel(n_valid, rows_ref, cols_ref, rout_ref, rstart_ref, fin_ref,
                ph1_ref, ph1f_ref, wout_ref, cmask_ref, adj_ref, x_ref,
                w1_ref, b1_ref, w2_ref, b2_ref, o_ref, h_ref, s2_ref,
                oacc_ref, s1_ref):
    t = pl.program_id(0)
    bm = adj_ref.shape[0]
    w = adj_ref.shape[1]
    r = rows_ref[t]
    c = cols_ref[t]
    valid_last = n_valid - (_NBC - 1) * w

    @pl.when(t == 0)
    def _build_s1():
        s1_ref[...] = jnp.zeros_like(s1_ref)
        s1_ref[pl.ds(0, n_valid), :] = jnp.dot(
            x_ref[...], w1_ref[...], preferred_element_type=jnp.float32)

    if valid_last < w:

        @pl.when(cmask_ref[t] == 1)
        def _zero_overhang_cols():
            adj_ref[:, pl.ds(valid_last, w - valid_last)] = jnp.zeros(
                (bm, w - valid_last), jnp.float32)

    @pl.when(rstart_ref[t] == 1)
    def _row_start():
        h_ref[...] = jnp.zeros_like(h_ref)

    @pl.when(t < _TA)
    def _phase0():
        h_ref[...] += jnp.dot(adj_ref[...], s1_ref[pl.ds(c * w, w), :],
                              preferred_element_type=jnp.float32)

    @pl.when(fin_ref[t] == 1)
    def _finalize_s2():
        h = jnp.maximum(h_ref[...] + b1_ref[...], 0.0)
        s2_blk = jnp.dot(h, w2_ref[...], preferred_element_type=jnp.float32)
        row_ids = r * bm + jax.lax.broadcasted_iota(jnp.int32, s2_blk.shape,
                                                    0)
        s2_ref[pl.ds(r * bm, bm), :] = jnp.where(row_ids < n_valid, s2_blk,
                                                 0.0)

    @pl.when(ph1_ref[t] == 1)
    def _phase1():
        contrib = jnp.dot(adj_ref[...], s2_ref[pl.ds(c * w, w), :],
                          preferred_element_type=jnp.float32)

        @pl.when(ph1f_ref[t] == 1)
        def _first():
            oacc_ref[pl.ds(r * bm, bm), :] = contrib

        @pl.when(ph1f_ref[t] == 0)
        def _rest():
            oacc_ref[pl.ds(r * bm, bm), :] += contrib

    @pl.when(wout_ref[t] == 1)
    def _writeout():
        o = oacc_ref[pl.ds(r * bm, bm), :] + b2_ref[...]
        m = jnp.max(o, axis=-1, keepdims=True)
        e = o - m
        lse = jnp.log(jnp.sum(jnp.exp(e), axis=-1, keepdims=True))
        o_ref[...] = e - lse


def kernel(x, adj, W1, b1, W2, b2):
    n, f_in = x.shape
    hid = W1.shape[1]
    out_f = W2.shape[1]
    per_blk = (n + _NBR - 1) // _NBR
    bm = ((per_blk + 127) // 128) * 128
    w = _K * bm
    npad = _NBR * bm

    b1r = b1.reshape(1, hid)
    b2r = b2.reshape(1, out_f)
    cmask = ((_SCHED[1] == _NBC - 1) & (n % w != 0)).astype(np.int32)
    sched = tuple(jnp.asarray(a) for a in _SCHED) + (jnp.asarray(cmask),)

    grid_spec = pltpu.PrefetchScalarGridSpec(
        num_scalar_prefetch=9,
        grid=(_T,),
        in_specs=[
            pl.BlockSpec((bm, w),
                         lambda t, rows, cols, *_: (rows[t], cols[t])),
            pl.BlockSpec((n, f_in), lambda t, *_: (0, 0)),
            pl.BlockSpec((f_in, hid), lambda t, *_: (0, 0)),
            pl.BlockSpec((1, hid), lambda t, *_: (0, 0)),
            pl.BlockSpec((hid, out_f), lambda t, *_: (0, 0)),
            pl.BlockSpec((1, out_f), lambda t, *_: (0, 0)),
        ],
        out_specs=pl.BlockSpec((bm, out_f),
                               lambda t, rows, cols, rout, *_: (rout[t], 0)),
        scratch_shapes=[
            pltpu.VMEM((bm, hid), jnp.float32),
            pltpu.VMEM((npad, out_f), jnp.float32),
            pltpu.VMEM((npad, out_f), jnp.float32),
            pltpu.VMEM((npad, hid), jnp.float32),
        ],
    )

    out = pl.pallas_call(
        functools.partial(_gcn_kernel, n),
        grid_spec=grid_spec,
        out_shape=jax.ShapeDtypeStruct((n, out_f), jnp.float32),
    )(*sched, adj, x, W1, b1r, W2, b2r)
    return out


# speedup vs baseline: 1.0835x; 1.0032x over previous
import functools

import numpy as np

import jax
import jax.numpy as jnp
from jax.experimental import pallas as pl
from jax.experimental.pallas import tpu as pltpu

_NBR = 5
_K = 1
_NBC = _NBR // _K


def _build_schedule(nbr, k):
    nbc = nbr // k
    comp = lambda c: (c + 1) * k - 1
    rows, cols, fin_s2, ph1, wout, rstart = [], [], [], [], [], []
    passb = []
    for r in range(nbr):
        later = [c for c in range(nbc) if comp(c) > r]
        ready = [c for c in range(nbc) if comp(c) < r]
        trick = [c for c in range(nbc) if comp(c) == r]
        order = later + ready + trick
        for j, c in enumerate(order):
            rows.append(r)
            cols.append(c)
            rstart.append(1 if j == 0 else 0)
            last = j == len(order) - 1
            fin_s2.append(1 if last else 0)
            eligible = c in ready or (c in trick and last)
            ph1.append(1 if eligible else 0)
            if not eligible:
                passb.append((r, c))
            wout.append(0)
    b_by_row = {}
    for r, c in passb:
        b_by_row.setdefault(r, []).append(c)
    for r in range(nbr):
        for j, c in enumerate(sorted(b_by_row.get(r, []))):
            rows.append(r)
            cols.append(c)
            rstart.append(0)
            fin_s2.append(0)
            ph1.append(1)
            wout.append(1 if j == len(b_by_row[r]) - 1 else 0)
    t_a = nbr * nbc
    for r in range(nbr):
        if r not in b_by_row:
            for t in range(t_a):
                if rows[t] == r and fin_s2[t]:
                    wout[t] = 1
    t_total = len(rows)
    seen = set()
    ph1f = [0] * t_total
    for t in range(t_total):
        if ph1[t] and rows[t] not in seen:
            seen.add(rows[t])
            ph1f[t] = 1
    rout = [0] * t_total
    nxt = rows[-1]
    for t in range(t_total - 1, -1, -1):
        if wout[t]:
            nxt = rows[t]
        rout[t] = nxt
    mk = lambda a: np.asarray(a, dtype=np.int32)
    return (mk(rows), mk(cols), mk(rout), mk(rstart), mk(fin_s2), mk(ph1),
            mk(ph1f), mk(wout)), t_a


_SCHED, _TA = _build_schedule(_NBR, _K)
_T = int(_SCHED[0].shape[0])


def _gcn_kernel(n_valid, rows_ref, cols_ref, rout_ref, rstart_ref, fin_ref,
                ph1_ref, ph1f_ref, wout_ref, cmask_ref, fuse_ref, adj_ref,
                x_ref, w1_ref, b1_ref, w2_ref, b2_ref, o_ref, h_ref,
                oacc_ref, comb_ref):
    t = pl.program_id(0)
    bm = adj_ref.shape[0]
    w = adj_ref.shape[1]
    hid = w1_ref.shape[1]
    out_f = w2_ref.shape[1]
    r = rows_ref[t]
    c = cols_ref[t]
    valid_last = n_valid - (_NBC - 1) * w

    @pl.when(t == 0)
    def _build_s1():
        comb_ref[...] = jnp.zeros_like(comb_ref)
        comb_ref[pl.ds(0, n_valid), pl.ds(0, hid)] = jnp.dot(
            x_ref[...], w1_ref[...], preferred_element_type=jnp.float32)

    def _acc_out(contrib):
        @pl.when(ph1f_ref[t] == 1)
        def _first():
            oacc_ref[pl.ds(r * bm, bm), :] = contrib

        @pl.when(ph1f_ref[t] == 0)
        def _rest():
            oacc_ref[pl.ds(r * bm, bm), :] += contrib

    if valid_last < w:

        @pl.when(cmask_ref[t] == 1)
        def _zero_overhang_cols():
            adj_ref[:, pl.ds(valid_last, w - valid_last)] = jnp.zeros(
                (bm, w - valid_last), jnp.float32)

    @pl.when(rstart_ref[t] == 1)
    def _row_start():
        h_ref[...] = jnp.zeros_like(h_ref)

    @pl.when(fuse_ref[t] == 1)
    def _fused_both():
        res = jnp.dot(adj_ref[...], comb_ref[pl.ds(c * w, w), :],
                      preferred_element_type=jnp.float32)
        h_ref[...] += res[:, 0:hid]
        _acc_out(res[:, hid:hid + out_f])

    @pl.when((t < _TA) & (fuse_ref[t] == 0))
    def _phase0():
        h_ref[...] += jnp.dot(adj_ref[...],
                              comb_ref[pl.ds(c * w, w), pl.ds(0, hid)],
                              preferred_element_type=jnp.float32)

    @pl.when(fin_ref[t] == 1)
    def _finalize_s2():
        h = jnp.maximum(h_ref[...] + b1_ref[...], 0.0)
        s2_blk = jnp.dot(h, w2_ref[...], preferred_element_type=jnp.float32)
        row_ids = r * bm + jax.lax.broadcasted_iota(jnp.int32, s2_blk.shape,
                                                    0)
        comb_ref[pl.ds(r * bm, bm), pl.ds(hid, out_f)] = jnp.where(
            row_ids < n_valid, s2_blk, 0.0)

    @pl.when((ph1_ref[t] == 1) & (fuse_ref[t] == 0))
    def _phase1():
        contrib = jnp.dot(adj_ref[...],
                          comb_ref[pl.ds(c * w, w), pl.ds(hid, out_f)],
                          preferred_element_type=jnp.float32)
        _acc_out(contrib)

    @pl.when(wout_ref[t] == 1)
    def _writeout():
        o = oacc_ref[pl.ds(r * bm, bm), :] + b2_ref[...]
        m = jnp.max(o, axis=-1, keepdims=True)
        e = o - m
        lse = jnp.log(jnp.sum(jnp.exp(e), axis=-1, keepdims=True))
        o_ref[...] = e - lse


def kernel(x, adj, W1, b1, W2, b2):
    n, f_in = x.shape
    hid = W1.shape[1]
    out_f = W2.shape[1]
    per_blk = (n + _NBR - 1) // _NBR
    bm = ((per_blk + 127) // 128) * 128
    w = _K * bm
    npad = _NBR * bm

    b1r = b1.reshape(1, hid)
    b2r = b2.reshape(1, out_f)
    cmask = ((_SCHED[1] == _NBC - 1) & (n % w != 0)).astype(np.int32)
    t_idx = np.arange(_T, dtype=np.int32)
    fuse = (_SCHED[5] & (1 - _SCHED[4]) & (t_idx < _TA)).astype(np.int32)
    sched = tuple(jnp.asarray(a) for a in _SCHED) + (jnp.asarray(cmask),
                                                     jnp.asarray(fuse))

    grid_spec = pltpu.PrefetchScalarGridSpec(
        num_scalar_prefetch=10,
        grid=(_T,),
        in_specs=[
            pl.BlockSpec((bm, w),
                         lambda t, rows, cols, *_: (rows[t], cols[t])),
            pl.BlockSpec((n, f_in), lambda t, *_: (0, 0)),
            pl.BlockSpec((f_in, hid), lambda t, *_: (0, 0)),
            pl.BlockSpec((1, hid), lambda t, *_: (0, 0)),
            pl.BlockSpec((hid, out_f), lambda t, *_: (0, 0)),
            pl.BlockSpec((1, out_f), lambda t, *_: (0, 0)),
        ],
        out_specs=pl.BlockSpec((bm, out_f),
                               lambda t, rows, cols, rout, *_: (rout[t], 0)),
        scratch_shapes=[
            pltpu.VMEM((bm, hid), jnp.float32),
            pltpu.VMEM((npad, out_f), jnp.float32),
            pltpu.VMEM((npad, hid + out_f), jnp.float32),
        ],
    )

    out = pl.pallas_call(
        functools.partial(_gcn_kernel, n),
        grid_spec=grid_spec,
        out_shape=jax.ShapeDtypeStruct((n, out_f), jnp.float32),
        compiler_params=pltpu.CompilerParams(
            vmem_limit_bytes=100 * 1024 * 1024),
    )(*sched, adj, x, W1, b1r, W2, b2r)
    return out
